# Initial kernel scaffold; baseline (speedup 1.0000x reference)
import jax
import jax.numpy as jnp
from jax.experimental import pallas as pl


def kernel(mh_output, delta_t_info, delta_dis_info, delta_rg_info, delta_entropy_info, city_embeddings, W_topk, b_topk, city):
    raise NotImplementedError("write your pallas kernel here")


# speedup vs baseline: 2.2210x; 2.2210x over previous
import functools

import jax
import jax.numpy as jnp
from jax.experimental import pallas as pl
from jax.experimental.pallas import tpu as pltpu

B, T, D = 4, 8192, 768
E = 8
TOP_K = 2
CITY_DIM = 32
N = B * T

BLK = 2048

NEG_INF = float("-inf")


def _router_body(mh_ref, dt_ref, dd_ref, rg_ref, de_ref, w_ref, cb_ref,
                 router_ref, idx_ref, gate_ref):
    acc = jnp.dot(mh_ref[...], w_ref[0:768, :], preferred_element_type=jnp.float32)
    acc += jnp.dot(dt_ref[...], w_ref[800:992, :], preferred_element_type=jnp.float32)
    acc += jnp.dot(dd_ref[...], w_ref[992:1184, :], preferred_element_type=jnp.float32)
    acc += jnp.dot(rg_ref[...], w_ref[1184:1280, :], preferred_element_type=jnp.float32)
    acc += jnp.dot(de_ref[...], w_ref[1280:1376, :], preferred_element_type=jnp.float32)
    logits = acc + cb_ref[...]

    lane = jax.lax.broadcasted_iota(jnp.int32, logits.shape, 1)

    m1 = jnp.max(logits, axis=-1, keepdims=True)
    ex = jnp.exp(logits - m1)
    gate_ref[...] = ex / jnp.sum(ex, axis=-1, keepdims=True)

    idx1 = jnp.min(jnp.where(logits == m1, lane, E), axis=-1, keepdims=True)
    is1 = lane == idx1
    l2 = jnp.where(is1, NEG_INF, logits)
    m2 = jnp.max(l2, axis=-1, keepdims=True)
    idx2 = jnp.min(jnp.where(l2 == m2, lane, E), axis=-1, keepdims=True)
    is2 = lane == idx2

    e2 = jnp.exp(m2 - m1)
    denom = 1.0 + e2
    p1 = 1.0 / denom
    p2 = e2 / denom
    router_ref[...] = jnp.where(is1, p1, jnp.where(is2, p2, 0.0))

    idx_ref[...] = jnp.concatenate([idx1, idx2], axis=-1).astype(jnp.int32)


@jax.jit
def _run(mh, dt, dd, rg, de, w, cb):
    grid = (N // BLK,)
    tok = lambda i: (i, 0)
    out = pl.pallas_call(
        _router_body,
        grid=grid,
        in_specs=[
            pl.BlockSpec((BLK, D), tok),
            pl.BlockSpec((BLK, D // 4), tok),
            pl.BlockSpec((BLK, D // 4), tok),
            pl.BlockSpec((BLK, D // 8), tok),
            pl.BlockSpec((BLK, D // 8), tok),
            pl.BlockSpec(w.shape, lambda i: (0, 0)),
            pl.BlockSpec(cb.shape, lambda i: (0, 0)),
        ],
        out_specs=[
            pl.BlockSpec((BLK, E), tok),
            pl.BlockSpec((BLK, TOP_K), tok),
            pl.BlockSpec((BLK, E), tok),
        ],
        out_shape=[
            jax.ShapeDtypeStruct((N, E), jnp.float32),
            jax.ShapeDtypeStruct((N, TOP_K), jnp.int32),
            jax.ShapeDtypeStruct((N, E), jnp.float32),
        ],
        compiler_params=pltpu.CompilerParams(
            dimension_semantics=("arbitrary",),
        ),
    )(mh, dt, dd, rg, de, w, cb)
    return out


def kernel(mh_output, delta_t_info, delta_dis_info, delta_rg_info,
           delta_entropy_info, city_embeddings, W_topk, b_topk, city):
    b, t, _ = mh_output.shape
    n = b * t
    mh = mh_output.reshape(n, D)
    dt = delta_t_info.reshape(n, D // 4)
    dd = delta_dis_info.reshape(n, D // 4)
    rg = delta_rg_info.reshape(n, D // 8)
    de = delta_entropy_info.reshape(n, D // 8)
    cb = (city_embeddings[city] @ W_topk[D:D + CITY_DIM, :] + b_topk).reshape(1, E)
    router, idx, gate = _run(mh, dt, dd, rg, de, W_topk, cb)
    return (router.reshape(b, t, E), idx.reshape(b, t, TOP_K),
            gate.reshape(b, t, E))


# speedup vs baseline: 2.2788x; 1.0260x over previous
import functools

import jax
import jax.numpy as jnp
from jax.experimental import pallas as pl
from jax.experimental.pallas import tpu as pltpu

B, T, D = 4, 8192, 768
E = 8
TOP_K = 2
CITY_DIM = 32
N = B * T

BLK = 2048

NEG_INF = float("-inf")


def _router_body(mh_ref, dt_ref, dd_ref, rg_ref, de_ref, w_ref, cb_ref,
                 router_ref, idx_ref, gate_ref):
    acc = jnp.dot(mh_ref[...], w_ref[0:768, :], preferred_element_type=jnp.float32)
    acc += jnp.dot(dt_ref[...], w_ref[800:992, :], preferred_element_type=jnp.float32)
    acc += jnp.dot(dd_ref[...], w_ref[992:1184, :], preferred_element_type=jnp.float32)
    acc += jnp.dot(rg_ref[...], w_ref[1184:1280, :], preferred_element_type=jnp.float32)
    acc += jnp.dot(de_ref[...], w_ref[1280:1376, :], preferred_element_type=jnp.float32)
    lt = jnp.transpose(acc + cb_ref[...])

    srow = jax.lax.broadcasted_iota(jnp.int32, lt.shape, 0).astype(jnp.float32)

    m1 = jnp.max(lt, axis=0, keepdims=True)
    ex = jnp.exp(lt - m1)
    gate_t = ex / jnp.sum(ex, axis=0, keepdims=True)

    idx1 = jnp.min(jnp.where(lt == m1, srow, float(E)), axis=0, keepdims=True)
    is1 = srow == idx1
    l2 = jnp.where(is1, NEG_INF, lt)
    m2 = jnp.max(l2, axis=0, keepdims=True)
    idx2 = jnp.min(jnp.where(l2 == m2, srow, float(E)), axis=0, keepdims=True)
    is2 = srow == idx2

    e2 = jnp.exp(m2 - m1)
    denom = 1.0 + e2
    p1 = jnp.broadcast_to(1.0 / denom, lt.shape)
    p2 = jnp.broadcast_to(e2 / denom, lt.shape)
    router_t = jnp.where(is1, p1, jnp.where(is2, p2, 0.0))

    router_ref[...] = jnp.transpose(router_t)
    gate_ref[...] = jnp.transpose(gate_t)
    packed = jnp.concatenate(
        [idx1, idx2, jnp.zeros((E - 2, lt.shape[1]), jnp.float32)], axis=0)
    idx_ref[...] = jnp.transpose(packed)[:, :TOP_K].astype(jnp.int32)


@jax.jit
def _run(mh, dt, dd, rg, de, w, cb):
    grid = (N // BLK,)
    tok = lambda i: (i, 0)
    out = pl.pallas_call(
        _router_body,
        grid=grid,
        in_specs=[
            pl.BlockSpec((BLK, D), tok),
            pl.BlockSpec((BLK, D // 4), tok),
            pl.BlockSpec((BLK, D // 4), tok),
            pl.BlockSpec((BLK, D // 8), tok),
            pl.BlockSpec((BLK, D // 8), tok),
            pl.BlockSpec(w.shape, lambda i: (0, 0)),
            pl.BlockSpec(cb.shape, lambda i: (0, 0)),
        ],
        out_specs=[
            pl.BlockSpec((BLK, E), tok),
            pl.BlockSpec((BLK, TOP_K), tok),
            pl.BlockSpec((BLK, E), tok),
        ],
        out_shape=[
            jax.ShapeDtypeStruct((N, E), jnp.float32),
            jax.ShapeDtypeStruct((N, TOP_K), jnp.int32),
            jax.ShapeDtypeStruct((N, E), jnp.float32),
        ],
        compiler_params=pltpu.CompilerParams(
            dimension_semantics=("arbitrary",),
        ),
    )(mh, dt, dd, rg, de, w, cb)
    return out


def kernel(mh_output, delta_t_info, delta_dis_info, delta_rg_info,
           delta_entropy_info, city_embeddings, W_topk, b_topk, city):
    b, t, _ = mh_output.shape
    n = b * t
    mh = mh_output.reshape(n, D)
    dt = delta_t_info.reshape(n, D // 4)
    dd = delta_dis_info.reshape(n, D // 4)
    rg = delta_rg_info.reshape(n, D // 8)
    de = delta_entropy_info.reshape(n, D // 8)
    cb = (city_embeddings[city] @ W_topk[D:D + CITY_DIM, :] + b_topk).reshape(1, E)
    router, idx, gate = _run(mh, dt, dd, rg, de, W_topk, cb)
    return (router.reshape(b, t, E), idx.reshape(b, t, TOP_K),
            gate.reshape(b, t, E))


# speedup vs baseline: 2.3781x; 1.0436x over previous
import jax
import jax.numpy as jnp
from jax.experimental import pallas as pl
from jax.experimental.pallas import tpu as pltpu

B, T, D = 4, 8192, 768
E = 8
TOP_K = 2
CITY_DIM = 32

BLK = 2048

NEG_INF = float("-inf")


def _router_body(mh_ref, dt_ref, dd_ref, rg_ref, de_ref, w_ref, cb_ref,
                 router_ref, idx_ref, gate_ref):
    acc = jnp.dot(mh_ref[0], w_ref[0:768, :], preferred_element_type=jnp.float32)
    acc += jnp.dot(dt_ref[0], w_ref[800:992, :], preferred_element_type=jnp.float32)
    acc += jnp.dot(dd_ref[0], w_ref[992:1184, :], preferred_element_type=jnp.float32)
    acc += jnp.dot(rg_ref[0], w_ref[1184:1280, :], preferred_element_type=jnp.float32)
    acc += jnp.dot(de_ref[0], w_ref[1280:1376, :], preferred_element_type=jnp.float32)

    lt = jnp.transpose(acc + cb_ref[...])

    srow = jax.lax.broadcasted_iota(jnp.int32, lt.shape, 0).astype(jnp.float32)

    m1 = jnp.max(lt, axis=0, keepdims=True)
    ex = jnp.exp(lt - m1)
    gate_t = ex / jnp.sum(ex, axis=0, keepdims=True)

    idx1 = jnp.min(jnp.where(lt == m1, srow, float(E)), axis=0, keepdims=True)
    is1 = srow == idx1
    l2 = jnp.where(is1, NEG_INF, lt)
    m2 = jnp.max(l2, axis=0, keepdims=True)
    idx2 = jnp.min(jnp.where(l2 == m2, srow, float(E)), axis=0, keepdims=True)
    is2 = srow == idx2

    e2 = jnp.exp(m2 - m1)
    denom = 1.0 + e2
    p1 = jnp.broadcast_to(1.0 / denom, lt.shape)
    p2 = jnp.broadcast_to(e2 / denom, lt.shape)
    router_t = jnp.where(is1, p1, jnp.where(is2, p2, 0.0))

    router_ref[...] = jnp.transpose(router_t)[None]
    gate_ref[...] = jnp.transpose(gate_t)[None]
    packed = jnp.concatenate(
        [idx1, idx2, jnp.zeros((E - 2, lt.shape[1]), jnp.float32)], axis=0)
    idx_ref[...] = jnp.transpose(packed)[None, :, :TOP_K].astype(jnp.int32)


@jax.jit
def _run(mh, dt, dd, rg, de, w, cb):
    grid = (B, T // BLK)
    tok = lambda b, i: (b, i, 0)
    fixed = lambda b, i: (0, 0)
    out = pl.pallas_call(
        _router_body,
        grid=grid,
        in_specs=[
            pl.BlockSpec((1, BLK, D), tok),
            pl.BlockSpec((1, BLK, D // 4), tok),
            pl.BlockSpec((1, BLK, D // 4), tok),
            pl.BlockSpec((1, BLK, D // 8), tok),
            pl.BlockSpec((1, BLK, D // 8), tok),
            pl.BlockSpec(w.shape, fixed),
            pl.BlockSpec(cb.shape, fixed),
        ],
        out_specs=[
            pl.BlockSpec((1, BLK, E), tok),
            pl.BlockSpec((1, BLK, TOP_K), tok),
            pl.BlockSpec((1, BLK, E), tok),
        ],
        out_shape=[
            jax.ShapeDtypeStruct((B, T, E), jnp.float32),
            jax.ShapeDtypeStruct((B, T, TOP_K), jnp.int32),
            jax.ShapeDtypeStruct((B, T, E), jnp.float32),
        ],
        compiler_params=pltpu.CompilerParams(
            dimension_semantics=("arbitrary", "arbitrary"),
        ),
    )(mh, dt, dd, rg, de, w, cb)
    return out


def kernel(mh_output, delta_t_info, delta_dis_info, delta_rg_info,
           delta_entropy_info, city_embeddings, W_topk, b_topk, city):
    cb = (city_embeddings[city] @ W_topk[D:D + CITY_DIM, :] + b_topk).reshape(1, E)
    return _run(mh_output, delta_t_info, delta_dis_info, delta_rg_info,
                delta_entropy_info, W_topk, cb)


# speedup vs baseline: 7.9255x; 3.3326x over previous
import jax
import jax.numpy as jnp
from jax.experimental import pallas as pl
from jax.experimental.pallas import tpu as pltpu

B, T, D = 4, 8192, 768
E = 8
TOP_K = 2
CITY_DIM = 32

BLK = 2048

NEG_INF = float("-inf")


def _router_body(mh_ref, dt_ref, dd_ref, rg_ref, de_ref, w_ref, cb_ref,
                 router_ref, idx_ref, gate_ref):
    acc = jnp.dot(mh_ref[0], w_ref[0:768, :], preferred_element_type=jnp.float32)
    lt = jnp.transpose(acc)
    lt += jnp.dot(jnp.transpose(w_ref[800:992, :]), dt_ref[0],
                  preferred_element_type=jnp.float32)
    lt += jnp.dot(jnp.transpose(w_ref[992:1184, :]), dd_ref[0],
                  preferred_element_type=jnp.float32)
    lt += jnp.dot(jnp.transpose(w_ref[1184:1280, :]), rg_ref[0],
                  preferred_element_type=jnp.float32)
    lt += jnp.dot(jnp.transpose(w_ref[1280:1376, :]), de_ref[0],
                  preferred_element_type=jnp.float32)
    lt += cb_ref[...]

    srow = jax.lax.broadcasted_iota(jnp.int32, lt.shape, 0).astype(jnp.float32)

    m1 = jnp.max(lt, axis=0, keepdims=True)
    ex = jnp.exp(lt - m1)
    gate_ref[...] = (ex / jnp.sum(ex, axis=0, keepdims=True))[None]

    idx1 = jnp.min(jnp.where(lt == m1, srow, float(E)), axis=0, keepdims=True)
    is1 = srow == idx1
    l2 = jnp.where(is1, NEG_INF, lt)
    m2 = jnp.max(l2, axis=0, keepdims=True)
    idx2 = jnp.min(jnp.where(l2 == m2, srow, float(E)), axis=0, keepdims=True)
    is2 = srow == idx2

    e2 = jnp.exp(m2 - m1)
    denom = 1.0 + e2
    p1 = jnp.broadcast_to(1.0 / denom, lt.shape)
    p2 = jnp.broadcast_to(e2 / denom, lt.shape)
    router_ref[...] = jnp.where(is1, p1, jnp.where(is2, p2, 0.0))[None]

    idx_ref[...] = jnp.concatenate([idx1, idx2], axis=0).astype(jnp.int32)[None]


@jax.jit
def _run(mh, dtT, ddT, rgT, deT, w, cb):
    grid = (B, T // BLK)
    tok = lambda b, i: (b, i, 0)
    feat = lambda b, i: (b, 0, i)
    fixed = lambda b, i: (0, 0)
    out = pl.pallas_call(
        _router_body,
        grid=grid,
        in_specs=[
            pl.BlockSpec((1, BLK, D), tok),
            pl.BlockSpec((1, D // 4, BLK), feat),
            pl.BlockSpec((1, D // 4, BLK), feat),
            pl.BlockSpec((1, D // 8, BLK), feat),
            pl.BlockSpec((1, D // 8, BLK), feat),
            pl.BlockSpec(w.shape, fixed),
            pl.BlockSpec(cb.shape, fixed),
        ],
        out_specs=[
            pl.BlockSpec((1, E, BLK), feat),
            pl.BlockSpec((1, TOP_K, BLK), feat),
            pl.BlockSpec((1, E, BLK), feat),
        ],
        out_shape=[
            jax.ShapeDtypeStruct((B, E, T), jnp.float32),
            jax.ShapeDtypeStruct((B, TOP_K, T), jnp.int32),
            jax.ShapeDtypeStruct((B, E, T), jnp.float32),
        ],
        compiler_params=pltpu.CompilerParams(
            dimension_semantics=("arbitrary", "arbitrary"),
        ),
    )(mh, dtT, ddT, rgT, deT, w, cb)
    return out


def kernel(mh_output, delta_t_info, delta_dis_info, delta_rg_info,
           delta_entropy_info, city_embeddings, W_topk, b_topk, city):
    cb = (city_embeddings[city] @ W_topk[D:D + CITY_DIM, :] + b_topk).reshape(E, 1)
    swap = lambda a: jnp.transpose(a, (0, 2, 1))
    routerT, idxT, gateT = _run(
        mh_output, swap(delta_t_info), swap(delta_dis_info),
        swap(delta_rg_info), swap(delta_entropy_info), W_topk, cb)
    return (swap(routerT), swap(idxT), swap(gateT))
